# Initial kernel scaffold; baseline (speedup 1.0000x reference)
import jax
import jax.numpy as jnp
from jax.experimental import pallas as pl


def kernel(x, edge_index, W1, b1, g1, be1, W2, b2, g2, be2, W3, b3, g3, be3, Wl, bl):
    raise NotImplementedError("write your pallas kernel here")


# speedup vs baseline: 24.2540x; 24.2540x over previous
import functools

import jax
import jax.numpy as jnp
from jax import lax
from jax.experimental import pallas as pl
from jax.experimental.pallas import tpu as pltpu
from jax.experimental.pallas import tpu_sc as plsc

NC = 2
NS = 16
NW = NC * NS
LANES = 16

N = 10000
E = 320000
C = 80
EW = E // NW
G = EW // C
NPAD = 10240
RPS = NPAD // NS
DRPS = NPAD // NS


def _sc_mesh():
    return plsc.VectorSubcoreMesh(
        core_axis_name="c", subcore_axis_name="s",
        num_cores=NC, num_subcores=NS)


def _deg_body(dst_hbm, out_hbm, dst_slab, ones_v, stage, acc, sem):
    c = lax.axis_index("c")
    s = lax.axis_index("s")
    wid = c * NS + s
    pltpu.sync_copy(dst_hbm.at[wid], dst_slab)

    def zrow(i, _):
        stage[pl.ds(i * LANES, LANES)] = jnp.zeros((LANES,), jnp.float32)
        return 0
    lax.fori_loop(0, DRPS // LANES, zrow, 0)

    def orow(i, _):
        ones_v[pl.ds(i * LANES, LANES)] = jnp.ones((LANES,), jnp.float32)
        return 0
    lax.fori_loop(0, C // LANES, orow, 0)

    pltpu.sync_copy(stage, acc.at[pl.ds(s * DRPS, DRPS)])
    plsc.subcore_barrier()

    def chunk(g, _):
        pltpu.sync_copy(ones_v, acc.at[dst_slab.at[g]], add=True)
        return 0
    lax.fori_loop(0, G, chunk, 0)
    plsc.subcore_barrier()

    pltpu.sync_copy(acc.at[pl.ds(s * DRPS, DRPS)], stage)
    pltpu.sync_copy(stage, out_hbm.at[pl.ds(wid * DRPS, DRPS)])


@functools.lru_cache(maxsize=None)
def _deg_kernel():
    return pl.kernel(
        _deg_body,
        out_type=jax.ShapeDtypeStruct((NC * NPAD,), jnp.float32),
        mesh=_sc_mesh(),
        scratch_types=[
            pltpu.VMEM((G, C), jnp.int32),
            pltpu.VMEM((C,), jnp.float32),
            pltpu.VMEM((DRPS,), jnp.float32),
            pltpu.VMEM_SHARED((NPAD,), jnp.float32),
            pltpu.SemaphoreType.DMA,
        ],
        compiler_params=pltpu.CompilerParams(use_tc_tiling_on_sc=False),
    )


def _deg_call(dst):
    return _deg_kernel()(dst)


def _agg_body(hs_hbm, src_hbm, dst_hbm, out_hbm,
              src_slab, dst_slab, rows, stage, acc, sem):
    c = lax.axis_index("c")
    s = lax.axis_index("s")
    wid = c * NS + s
    pltpu.sync_copy(src_hbm.at[wid], src_slab)
    pltpu.sync_copy(dst_hbm.at[wid], dst_slab)

    def zrow(i, _):
        stage[i] = jnp.zeros((LANES,), jnp.float32)
        return 0
    lax.fori_loop(0, RPS, zrow, 0)
    pltpu.sync_copy(stage, acc.at[pl.ds(s * RPS, RPS)])
    plsc.subcore_barrier()

    def chunk(g, _):
        pltpu.async_copy(hs_hbm.at[src_slab.at[g]], rows, sem).wait()
        pltpu.sync_copy(rows, acc.at[dst_slab.at[g]], add=True)
        return 0
    lax.fori_loop(0, G, chunk, 0)
    plsc.subcore_barrier()

    pltpu.sync_copy(acc.at[pl.ds(s * RPS, RPS)], stage)
    pltpu.sync_copy(stage, out_hbm.at[pl.ds(wid * RPS, RPS)])


@functools.lru_cache(maxsize=None)
def _agg_kernel():
    return pl.kernel(
        _agg_body,
        out_type=jax.ShapeDtypeStruct((NC * NPAD, LANES), jnp.float32),
        mesh=_sc_mesh(),
        scratch_types=[
            pltpu.VMEM((G, C), jnp.int32),
            pltpu.VMEM((G, C), jnp.int32),
            pltpu.VMEM((C, LANES), jnp.float32),
            pltpu.VMEM((RPS, LANES), jnp.float32),
            pltpu.VMEM_SHARED((NPAD, LANES), jnp.float32),
            pltpu.SemaphoreType.DMA,
        ],
        compiler_params=pltpu.CompilerParams(use_tc_tiling_on_sc=False),
    )


def _agg_call(hs, src, dst):
    return _agg_kernel()(hs, src, dst)


def _tc1_body(d0_ref, d1_ref, x_ref, w_ref, dinv_ref, hs_ref):
    deg = d0_ref[:N, :] + d1_ref[:N, :] + 1.0
    dinv = lax.rsqrt(deg)
    h = jnp.dot(x_ref[...], w_ref[...], preferred_element_type=jnp.float32)
    dinv_ref[...] = dinv
    hs_ref[...] = h * dinv


def _tc1(d0, d1, x, w):
    return pl.pallas_call(
        _tc1_body,
        out_shape=[
            jax.ShapeDtypeStruct((N, 1), jnp.float32),
            jax.ShapeDtypeStruct((N, LANES), jnp.float32),
        ],
    )(d0, d1, x, w)


def _ln_relu(t, g_ref, be_ref):
    mu = jnp.mean(t, axis=-1, keepdims=True)
    var = jnp.mean((t - mu) ** 2, axis=-1, keepdims=True)
    tn = (t - mu) * lax.rsqrt(var + 1e-5) * g_ref[...][None, :] + be_ref[...][None, :]
    return jnp.maximum(tn, 0.0)


def _tc_mid_body(a0_ref, a1_ref, hs_ref, dinv_ref, b_ref, g_ref, be_ref,
                 w_ref, out_ref):
    dinv = dinv_ref[...]
    t = (a0_ref[...] + a1_ref[...] + hs_ref[...]) * dinv + b_ref[...][None, :]
    r = _ln_relu(t, g_ref, be_ref)
    out_ref[...] = jnp.dot(r, w_ref[...],
                           preferred_element_type=jnp.float32) * dinv


def _tc_mid(a0, a1, hs, dinv, b, g, be, w_next):
    return pl.pallas_call(
        _tc_mid_body,
        out_shape=jax.ShapeDtypeStruct((N, LANES), jnp.float32),
    )(a0, a1, hs, dinv, b, g, be, w_next)


def _tc_final_body(a0_ref, a1_ref, hs_ref, dinv_ref, b_ref, g_ref, be_ref,
                   wl_ref, bl_ref, out_ref):
    t = (a0_ref[...] + a1_ref[...] + hs_ref[...]) * dinv_ref[...] \
        + b_ref[...][None, :]
    r = _ln_relu(t, g_ref, be_ref)
    pooled = jnp.mean(r, axis=0, keepdims=True)
    out_ref[...] = jnp.dot(pooled, wl_ref[...],
                           preferred_element_type=jnp.float32) \
        + bl_ref[...][None, :]


def _tc_final(a0, a1, hs, dinv, b, g, be, wl, bl):
    return pl.pallas_call(
        _tc_final_body,
        out_shape=jax.ShapeDtypeStruct((1, 25), jnp.float32),
    )(a0, a1, hs, dinv, b, g, be, wl, bl)


def kernel(x, edge_index, W1, b1, g1, be1, W2, b2, g2, be2, W3, b3, g3, be3,
           Wl, bl):
    src = edge_index[0].reshape(NW, G, C)
    dst = edge_index[1].reshape(NW, G, C)

    deg_parts = _deg_call(dst)
    d0 = deg_parts[:NPAD].reshape(NPAD, 1)
    d1 = deg_parts[NPAD:].reshape(NPAD, 1)
    dinv, hs = _tc1(d0, d1, x, W1)

    agg = _agg_call(hs, src, dst)
    hs = _tc_mid(agg[:N], agg[NPAD:NPAD + N], hs, dinv, b1, g1, be1, W2)

    agg = _agg_call(hs, src, dst)
    hs = _tc_mid(agg[:N], agg[NPAD:NPAD + N], hs, dinv, b2, g2, be2, W3)

    agg = _agg_call(hs, src, dst)
    out = _tc_final(agg[:N], agg[NPAD:NPAD + N], hs, dinv, b3, g3, be3,
                    Wl, bl)
    return out.reshape(25)


# speedup vs baseline: 36.6051x; 1.5092x over previous
import functools

import jax
import jax.numpy as jnp
from jax import lax
from jax.experimental import pallas as pl
from jax.experimental.pallas import tpu as pltpu
from jax.experimental.pallas import tpu_sc as plsc

NC = 2
NS = 16
NW = NC * NS
LANES = 16

N = 10000
E = 320000
C = 80
EW = E // NW
G = EW // C
NPAD = 10240
RPS = NPAD // NS
DRPS = NPAD // NS


def _sc_mesh():
    return plsc.VectorSubcoreMesh(
        core_axis_name="c", subcore_axis_name="s",
        num_cores=NC, num_subcores=NS)


def _deg_body(dst_hbm, out_hbm, dst_slab, ones_v, stage, acc, sem):
    c = lax.axis_index("c")
    s = lax.axis_index("s")
    wid = c * NS + s
    pltpu.sync_copy(dst_hbm.at[wid], dst_slab)

    def zrow(i, _):
        stage[pl.ds(i * LANES, LANES)] = jnp.zeros((LANES,), jnp.float32)
        return 0
    lax.fori_loop(0, DRPS // LANES, zrow, 0)

    def orow(i, _):
        ones_v[pl.ds(i * LANES, LANES)] = jnp.ones((LANES,), jnp.float32)
        return 0
    lax.fori_loop(0, C // LANES, orow, 0)

    pltpu.sync_copy(stage, acc.at[pl.ds(s * DRPS, DRPS)])
    plsc.subcore_barrier()

    def chunk(g, _):
        pltpu.sync_copy(ones_v, acc.at[dst_slab.at[g]], add=True)
        return 0
    lax.fori_loop(0, G, chunk, 0)
    plsc.subcore_barrier()

    pltpu.sync_copy(acc.at[pl.ds(s * DRPS, DRPS)], stage)
    pltpu.sync_copy(stage, out_hbm.at[pl.ds(wid * DRPS, DRPS)])


@functools.lru_cache(maxsize=None)
def _deg_kernel():
    return pl.kernel(
        _deg_body,
        out_type=jax.ShapeDtypeStruct((NC * NPAD,), jnp.float32),
        mesh=_sc_mesh(),
        scratch_types=[
            pltpu.VMEM((G, C), jnp.int32),
            pltpu.VMEM((C,), jnp.float32),
            pltpu.VMEM((DRPS,), jnp.float32),
            pltpu.VMEM_SHARED((NPAD,), jnp.float32),
            pltpu.SemaphoreType.DMA,
        ],
        compiler_params=pltpu.CompilerParams(use_tc_tiling_on_sc=False),
    )


def _deg_call(dst):
    return _deg_kernel()(dst)


NB = 4
PD = 2


def _agg_body(hs_hbm, src_hbm, dst_hbm, out_hbm,
              src_slab, dst_slab, rows, stage, acc, gsem, ssem):
    c = lax.axis_index("c")
    s = lax.axis_index("s")
    wid = c * NS + s
    pltpu.sync_copy(src_hbm.at[wid], src_slab)
    pltpu.sync_copy(dst_hbm.at[wid], dst_slab)

    def zrow(i, _):
        stage[i] = jnp.zeros((LANES,), jnp.float32)
        return 0
    lax.fori_loop(0, RPS, zrow, 0)
    pltpu.sync_copy(stage, acc.at[pl.ds(s * RPS, RPS)])
    plsc.subcore_barrier()

    def fire_gather(g):
        b = lax.rem(g, NB)
        pltpu.async_copy(hs_hbm.at[src_slab.at[g]], rows.at[b], gsem.at[b])

    def drain(sem_arr, b):
        pltpu.make_async_copy(hs_hbm.at[pl.ds(0, C)], rows.at[b],
                              sem_arr.at[b]).wait()

    for g in range(PD):
        fire_gather(jnp.int32(g))

    def chunk(g, _):
        b = lax.rem(g, NB)
        drain(gsem, b)
        pltpu.async_copy(rows.at[b], acc.at[dst_slab.at[g]], ssem.at[b],
                         add=True)
        gp = g + PD
        bp = lax.rem(gp, NB)

        @pl.when(gp < G)
        def _():
            @pl.when(gp >= NB)
            def _():
                drain(ssem, bp)
            fire_gather(gp)
        return 0
    lax.fori_loop(0, G, chunk, 0)
    for k in range(NB):
        drain(ssem, jnp.int32((G - NB + k) % NB))
    plsc.subcore_barrier()

    pltpu.sync_copy(acc.at[pl.ds(s * RPS, RPS)], stage)
    pltpu.sync_copy(stage, out_hbm.at[pl.ds(wid * RPS, RPS)])


@functools.lru_cache(maxsize=None)
def _agg_kernel():
    return pl.kernel(
        _agg_body,
        out_type=jax.ShapeDtypeStruct((NC * NPAD, LANES), jnp.float32),
        mesh=_sc_mesh(),
        scratch_types=[
            pltpu.VMEM((G, C), jnp.int32),
            pltpu.VMEM((G, C), jnp.int32),
            pltpu.VMEM((NB, C, LANES), jnp.float32),
            pltpu.VMEM((RPS, LANES), jnp.float32),
            pltpu.VMEM_SHARED((NPAD, LANES), jnp.float32),
            pltpu.SemaphoreType.DMA((NB,)),
            pltpu.SemaphoreType.DMA((NB,)),
        ],
        compiler_params=pltpu.CompilerParams(use_tc_tiling_on_sc=False),
    )


def _agg_call(hs, src, dst):
    return _agg_kernel()(hs, src, dst)


def _tc1_body(d0_ref, d1_ref, x_ref, w_ref, dinv_ref, hs_ref):
    deg = d0_ref[:N, :] + d1_ref[:N, :] + 1.0
    dinv = lax.rsqrt(deg)
    h = jnp.dot(x_ref[...], w_ref[...], preferred_element_type=jnp.float32)
    dinv_ref[...] = dinv
    hs_ref[...] = h * dinv


def _tc1(d0, d1, x, w):
    return pl.pallas_call(
        _tc1_body,
        out_shape=[
            jax.ShapeDtypeStruct((N, 1), jnp.float32),
            jax.ShapeDtypeStruct((N, LANES), jnp.float32),
        ],
    )(d0, d1, x, w)


def _ln_relu(t, g_ref, be_ref):
    mu = jnp.mean(t, axis=-1, keepdims=True)
    var = jnp.mean((t - mu) ** 2, axis=-1, keepdims=True)
    tn = (t - mu) * lax.rsqrt(var + 1e-5) * g_ref[...][None, :] + be_ref[...][None, :]
    return jnp.maximum(tn, 0.0)


def _tc_mid_body(a0_ref, a1_ref, hs_ref, dinv_ref, b_ref, g_ref, be_ref,
                 w_ref, out_ref):
    dinv = dinv_ref[...]
    t = (a0_ref[...] + a1_ref[...] + hs_ref[...]) * dinv + b_ref[...][None, :]
    r = _ln_relu(t, g_ref, be_ref)
    out_ref[...] = jnp.dot(r, w_ref[...],
                           preferred_element_type=jnp.float32) * dinv


def _tc_mid(a0, a1, hs, dinv, b, g, be, w_next):
    return pl.pallas_call(
        _tc_mid_body,
        out_shape=jax.ShapeDtypeStruct((N, LANES), jnp.float32),
    )(a0, a1, hs, dinv, b, g, be, w_next)


def _tc_final_body(a0_ref, a1_ref, hs_ref, dinv_ref, b_ref, g_ref, be_ref,
                   wl_ref, bl_ref, out_ref):
    t = (a0_ref[...] + a1_ref[...] + hs_ref[...]) * dinv_ref[...] \
        + b_ref[...][None, :]
    r = _ln_relu(t, g_ref, be_ref)
    pooled = jnp.mean(r, axis=0, keepdims=True)
    out_ref[...] = jnp.dot(pooled, wl_ref[...],
                           preferred_element_type=jnp.float32) \
        + bl_ref[...][None, :]


def _tc_final(a0, a1, hs, dinv, b, g, be, wl, bl):
    return pl.pallas_call(
        _tc_final_body,
        out_shape=jax.ShapeDtypeStruct((1, 25), jnp.float32),
    )(a0, a1, hs, dinv, b, g, be, wl, bl)


def kernel(x, edge_index, W1, b1, g1, be1, W2, b2, g2, be2, W3, b3, g3, be3,
           Wl, bl):
    src = edge_index[0].reshape(NW, G, C)
    dst = edge_index[1].reshape(NW, G, C)

    deg_parts = _deg_call(dst)
    d0 = deg_parts[:NPAD].reshape(NPAD, 1)
    d1 = deg_parts[NPAD:].reshape(NPAD, 1)
    dinv, hs = _tc1(d0, d1, x, W1)

    agg = _agg_call(hs, src, dst)
    hs = _tc_mid(agg[:N], agg[NPAD:NPAD + N], hs, dinv, b1, g1, be1, W2)

    agg = _agg_call(hs, src, dst)
    hs = _tc_mid(agg[:N], agg[NPAD:NPAD + N], hs, dinv, b2, g2, be2, W3)

    agg = _agg_call(hs, src, dst)
    out = _tc_final(agg[:N], agg[NPAD:NPAD + N], hs, dinv, b3, g3, be3,
                    Wl, bl)
    return out.reshape(25)


# speedup vs baseline: 52.6257x; 1.4377x over previous
import functools

import jax
import jax.numpy as jnp
from jax import lax
from jax.experimental import pallas as pl
from jax.experimental.pallas import tpu as pltpu
from jax.experimental.pallas import tpu_sc as plsc

NC = 2
NS = 16
NW = NC * NS
LANES = 16

N = 10000
E = 320000
C = 80
EW = E // NW
G = EW // C
NPAD = 10240
RPS = NPAD // NS
NROW = N // 8
PPAD = NPAD // 8

NB = 4
PD = 2


def _sc_mesh():
    return plsc.VectorSubcoreMesh(
        core_axis_name="c", subcore_axis_name="s",
        num_cores=NC, num_subcores=NS)


def _deg_body(e_hbm, out_hbm, dst_slab, ones_v, stage, acc, sem):
    c = lax.axis_index("c")
    s = lax.axis_index("s")
    wid = c * NS + s
    pltpu.sync_copy(e_hbm.at[1].at[wid], dst_slab)

    def zrow(i, _):
        stage[i] = jnp.zeros((LANES,), jnp.float32)
        return 0
    lax.fori_loop(0, RPS, zrow, 0)

    def orow(i, _):
        ones_v[i] = jnp.ones((LANES,), jnp.float32)
        return 0
    lax.fori_loop(0, C, orow, 0)

    pltpu.sync_copy(stage, acc.at[pl.ds(s * RPS, RPS)])
    plsc.subcore_barrier()

    def drain():
        pltpu.make_async_copy(out_hbm.at[pl.ds(0, C)], ones_v, sem).wait()

    def chunk(g, _):
        pltpu.async_copy(ones_v, acc.at[dst_slab.at[g]], sem, add=True)

        @pl.when(g >= NB)
        def _():
            drain()
        return 0
    lax.fori_loop(0, G, chunk, 0)
    for _ in range(NB):
        drain()
    plsc.subcore_barrier()

    pltpu.sync_copy(acc.at[pl.ds(s * RPS, RPS)], stage)
    pltpu.sync_copy(stage, out_hbm.at[pl.ds(wid * RPS, RPS)])


@functools.lru_cache(maxsize=None)
def _deg_kernel():
    return pl.kernel(
        _deg_body,
        out_type=jax.ShapeDtypeStruct((NC * NPAD, LANES), jnp.float32),
        mesh=_sc_mesh(),
        scratch_types=[
            pltpu.VMEM((G, C), jnp.int32),
            pltpu.VMEM((C, LANES), jnp.float32),
            pltpu.VMEM((RPS, LANES), jnp.float32),
            pltpu.VMEM_SHARED((NPAD, LANES), jnp.float32),
            pltpu.SemaphoreType.DMA,
        ],
        compiler_params=pltpu.CompilerParams(use_tc_tiling_on_sc=False),
    )


def _deg_call(e3):
    return _deg_kernel()(e3)


def _agg_body(hs_hbm, e_hbm, out_hbm,
              src_slab, dst_slab, rows, stage, acc, gsem, ssem):
    c = lax.axis_index("c")
    s = lax.axis_index("s")
    wid = c * NS + s
    pltpu.sync_copy(e_hbm.at[0].at[wid], src_slab)
    pltpu.sync_copy(e_hbm.at[1].at[wid], dst_slab)

    def zrow(i, _):
        stage[i] = jnp.zeros((LANES,), jnp.float32)
        return 0
    lax.fori_loop(0, RPS, zrow, 0)
    pltpu.sync_copy(stage, acc.at[pl.ds(s * RPS, RPS)])
    plsc.subcore_barrier()

    def fire_gather(g):
        b = lax.rem(g, NB)
        pltpu.async_copy(hs_hbm.at[src_slab.at[g]], rows.at[b], gsem.at[b])

    def drain(sem_arr, b):
        pltpu.make_async_copy(hs_hbm.at[pl.ds(0, C)], rows.at[b],
                              sem_arr.at[b]).wait()

    for g in range(PD):
        fire_gather(jnp.int32(g))

    def chunk(g, _):
        b = lax.rem(g, NB)
        drain(gsem, b)
        pltpu.async_copy(rows.at[b], acc.at[dst_slab.at[g]], ssem.at[b],
                         add=True)
        gp = g + PD
        bp = lax.rem(gp, NB)

        @pl.when(gp < G)
        def _():
            @pl.when(gp >= NB)
            def _():
                drain(ssem, bp)
            fire_gather(gp)
        return 0
    lax.fori_loop(0, G, chunk, 0)
    for k in range(NB):
        drain(ssem, jnp.int32((G - NB + k) % NB))
    plsc.subcore_barrier()

    pltpu.sync_copy(acc.at[pl.ds(s * RPS, RPS)], stage)
    pltpu.sync_copy(stage, out_hbm.at[pl.ds(wid * RPS, RPS)])


@functools.lru_cache(maxsize=None)
def _agg_kernel():
    return pl.kernel(
        _agg_body,
        out_type=jax.ShapeDtypeStruct((NC * NPAD, LANES), jnp.float32),
        mesh=_sc_mesh(),
        scratch_types=[
            pltpu.VMEM((G, C), jnp.int32),
            pltpu.VMEM((G, C), jnp.int32),
            pltpu.VMEM((NB, C, LANES), jnp.float32),
            pltpu.VMEM((RPS, LANES), jnp.float32),
            pltpu.VMEM_SHARED((NPAD, LANES), jnp.float32),
            pltpu.SemaphoreType.DMA((NB,)),
            pltpu.SemaphoreType.DMA((NB,)),
        ],
        compiler_params=pltpu.CompilerParams(use_tc_tiling_on_sc=False),
    )


def _agg_call(hs, e3):
    return _agg_kernel()(hs, e3)


def _tc1_body(degp_ref, x3_ref, w_ref, dinv_ref, hs_ref):
    deg = degp_ref[0:NROW, :] + degp_ref[PPAD:PPAD + NROW, :] + 1.0
    dinv = lax.rsqrt(deg)
    dinv_ref[...] = dinv
    for a in range(8):
        h = jnp.dot(x3_ref[:, a, :], w_ref[...],
                    preferred_element_type=jnp.float32)
        hs_ref[:, 16 * a:16 * (a + 1)] = h * dinv[:, 16 * a:16 * (a + 1)]


def _tc1(degp, x3, w):
    return pl.pallas_call(
        _tc1_body,
        out_shape=[
            jax.ShapeDtypeStruct((NROW, 128), jnp.float32),
            jax.ShapeDtypeStruct((NROW, 128), jnp.float32),
        ],
    )(degp, x3, w)


def _ln_relu(t, mavg_ref, gp_ref, bep_ref):
    hi = lax.Precision.HIGHEST
    mavg = mavg_ref[...]
    mu = jnp.dot(t, mavg, precision=hi, preferred_element_type=jnp.float32)
    d = t - mu
    var = jnp.dot(d * d, mavg, precision=hi,
                  preferred_element_type=jnp.float32)
    tn = d * lax.rsqrt(var + 1e-5) * gp_ref[...][None, :] \
        + bep_ref[...][None, :]
    return jnp.maximum(tn, 0.0)


def _tc_mid_body(aggp_ref, hs_ref, dinv_ref, bp_ref, gp_ref, bep_ref,
                 wb_ref, mavg_ref, out_ref):
    dinv = dinv_ref[...]
    t = (aggp_ref[0:NROW, :] + aggp_ref[PPAD:PPAD + NROW, :] + hs_ref[...]) \
        * dinv + bp_ref[...][None, :]
    r = _ln_relu(t, mavg_ref, gp_ref, bep_ref)
    out_ref[...] = jnp.dot(r, wb_ref[...],
                           preferred_element_type=jnp.float32) * dinv


def _tc_mid(aggp, hs, dinv, bp, gp, bep, wb, mavg):
    return pl.pallas_call(
        _tc_mid_body,
        out_shape=jax.ShapeDtypeStruct((NROW, 128), jnp.float32),
    )(aggp, hs, dinv, bp, gp, bep, wb, mavg)


def _tc_final_body(aggp_ref, hs_ref, dinv_ref, bp_ref, gp_ref, bep_ref,
                   mavg_ref, f_ref, wl_ref, bl_ref, out_ref):
    hi = lax.Precision.HIGHEST
    t = (aggp_ref[0:NROW, :] + aggp_ref[PPAD:PPAD + NROW, :] + hs_ref[...]) \
        * dinv_ref[...] + bp_ref[...][None, :]
    r = _ln_relu(t, mavg_ref, gp_ref, bep_ref)
    srow = jnp.sum(r, axis=0, keepdims=True)
    pooled = jnp.dot(srow, f_ref[...], precision=hi,
                     preferred_element_type=jnp.float32) * (1.0 / N)
    out_ref[...] = jnp.dot(pooled, wl_ref[...],
                           preferred_element_type=jnp.float32) \
        + bl_ref[...][None, :]


def _tc_final(aggp, hs, dinv, bp, gp, bep, mavg, f, wl, bl):
    return pl.pallas_call(
        _tc_final_body,
        out_shape=jax.ShapeDtypeStruct((1, 25), jnp.float32),
    )(aggp, hs, dinv, bp, gp, bep, mavg, f, wl, bl)


def kernel(x, edge_index, W1, b1, g1, be1, W2, b2, g2, be2, W3, b3, g3, be3,
           Wl, bl):
    e3 = edge_index.reshape(2, NW, G, C)
    x3 = x.reshape(NROW, 8, 128)
    eye8 = jnp.eye(8, dtype=jnp.float32)
    mavg = jnp.kron(eye8, jnp.full((16, 16), 1.0 / 16, jnp.float32))
    fmat = jnp.kron(jnp.ones((8, 1), jnp.float32),
                    jnp.eye(16, dtype=jnp.float32))

    def rep(v):
        return jnp.tile(v, 8)

    degp = _deg_call(e3).reshape(NC * PPAD, 128)
    dinv, hs = _tc1(degp, x3, W1)

    aggp = _agg_call(hs.reshape(N, LANES), e3).reshape(NC * PPAD, 128)
    hs = _tc_mid(aggp, hs, dinv, rep(b1), rep(g1), rep(be1),
                 jnp.kron(eye8, W2), mavg)

    aggp = _agg_call(hs.reshape(N, LANES), e3).reshape(NC * PPAD, 128)
    hs = _tc_mid(aggp, hs, dinv, rep(b2), rep(g2), rep(be2),
                 jnp.kron(eye8, W3), mavg)

    aggp = _agg_call(hs.reshape(N, LANES), e3).reshape(NC * PPAD, 128)
    out = _tc_final(aggp, hs, dinv, rep(b3), rep(g3), rep(be3),
                    mavg, fmat, Wl, bl)
    return out.reshape(25)


# speedup vs baseline: 63.6368x; 1.2092x over previous
import functools

import jax
import jax.numpy as jnp
from jax import lax
from jax.experimental import pallas as pl
from jax.experimental.pallas import tpu as pltpu
from jax.experimental.pallas import tpu_sc as plsc

NC = 2
NS = 16
NW = NC * NS
LANES = 16

N = 10000
E = 320000
C = 80
EW = E // NW
G = EW // C
NPAD = 10240
RPS = NPAD // NS
NROW = N // 8
PPAD = NPAD // 8

NB = 6
PD = 3


def _sc_mesh():
    return plsc.VectorSubcoreMesh(
        core_axis_name="c", subcore_axis_name="s",
        num_cores=NC, num_subcores=NS)


def _deg_body(e_hbm, out_hbm, dst_slab, ones_v, stage, acc, sem):
    c = lax.axis_index("c")
    s = lax.axis_index("s")
    wid = c * NS + s
    pltpu.sync_copy(e_hbm.at[1].at[wid], dst_slab)

    def zrow(i, _):
        stage[i] = jnp.zeros((LANES,), jnp.float32)
        return 0
    lax.fori_loop(0, RPS, zrow, 0)

    def orow(i, _):
        ones_v[i] = jnp.ones((LANES,), jnp.float32)
        return 0
    lax.fori_loop(0, C, orow, 0)

    pltpu.sync_copy(stage, acc.at[pl.ds(s * RPS, RPS)])
    plsc.subcore_barrier()

    def drain():
        pltpu.make_async_copy(out_hbm.at[pl.ds(0, C)], ones_v, sem).wait()

    def chunk(g, _):
        pltpu.async_copy(ones_v, acc.at[dst_slab.at[g]], sem, add=True)

        @pl.when(g >= NB)
        def _():
            drain()
        return 0
    lax.fori_loop(0, G, chunk, 0)
    for _ in range(NB):
        drain()
    plsc.subcore_barrier()

    pltpu.sync_copy(acc.at[pl.ds(s * RPS, RPS)], stage)
    pltpu.sync_copy(stage, out_hbm.at[pl.ds(wid * RPS, RPS)])


@functools.lru_cache(maxsize=None)
def _deg_kernel():
    return pl.kernel(
        _deg_body,
        out_type=jax.ShapeDtypeStruct((NC * NPAD, LANES), jnp.float32),
        mesh=_sc_mesh(),
        scratch_types=[
            pltpu.VMEM((G, C), jnp.int32),
            pltpu.VMEM((C, LANES), jnp.float32),
            pltpu.VMEM((RPS, LANES), jnp.float32),
            pltpu.VMEM_SHARED((NPAD, LANES), jnp.float32),
            pltpu.SemaphoreType.DMA,
        ],
        compiler_params=pltpu.CompilerParams(use_tc_tiling_on_sc=False),
    )


def _deg_call(e3):
    return _deg_kernel()(e3)


def _agg_body(hs_hbm, e_hbm, out_hbm,
              src_slab, dst_slab, rows, stage, acc, gsem, ssem):
    c = lax.axis_index("c")
    s = lax.axis_index("s")
    wid = c * NS + s
    pltpu.sync_copy(e_hbm.at[0].at[wid], src_slab)
    pltpu.sync_copy(e_hbm.at[1].at[wid], dst_slab)

    def zrow(i, _):
        stage[i] = jnp.zeros((LANES,), jnp.float32)
        return 0
    lax.fori_loop(0, RPS, zrow, 0)
    pltpu.sync_copy(stage, acc.at[pl.ds(s * RPS, RPS)])
    plsc.subcore_barrier()

    def fire_gather(g):
        b = lax.rem(g, NB)
        pltpu.async_copy(hs_hbm.at[src_slab.at[g]], rows.at[b], gsem.at[b])

    def drain(sem_arr, b):
        pltpu.make_async_copy(hs_hbm.at[pl.ds(0, C)], rows.at[b],
                              sem_arr.at[b]).wait()

    for g in range(PD):
        fire_gather(jnp.int32(g))

    def chunk(g, _):
        b = lax.rem(g, NB)
        drain(gsem, b)
        pltpu.async_copy(rows.at[b], acc.at[dst_slab.at[g]], ssem.at[b],
                         add=True)
        gp = g + PD
        bp = lax.rem(gp, NB)

        @pl.when(gp < G)
        def _():
            @pl.when(gp >= NB)
            def _():
                drain(ssem, bp)
            fire_gather(gp)
        return 0
    lax.fori_loop(0, G, chunk, 0)
    for k in range(NB):
        drain(ssem, jnp.int32((G - NB + k) % NB))
    plsc.subcore_barrier()

    pltpu.sync_copy(acc.at[pl.ds(s * RPS, RPS)], stage)
    pltpu.sync_copy(stage, out_hbm.at[pl.ds(wid * RPS, RPS)])


@functools.lru_cache(maxsize=None)
def _agg_kernel():
    return pl.kernel(
        _agg_body,
        out_type=jax.ShapeDtypeStruct((NC * NPAD, LANES), jnp.float32),
        mesh=_sc_mesh(),
        scratch_types=[
            pltpu.VMEM((G, C), jnp.int32),
            pltpu.VMEM((G, C), jnp.int32),
            pltpu.VMEM((NB, C, LANES), jnp.float32),
            pltpu.VMEM((RPS, LANES), jnp.float32),
            pltpu.VMEM_SHARED((NPAD, LANES), jnp.float32),
            pltpu.SemaphoreType.DMA((NB,)),
            pltpu.SemaphoreType.DMA((NB,)),
        ],
        compiler_params=pltpu.CompilerParams(use_tc_tiling_on_sc=False),
    )


def _agg_call(hs, e3):
    return _agg_kernel()(hs, e3)


def _tc1_body(degp_ref, x3_ref, w_ref, dinv_ref, hs_ref):
    deg = degp_ref[0:NROW, :] + degp_ref[PPAD:PPAD + NROW, :] + 1.0
    dinv = lax.rsqrt(deg)
    dinv_ref[...] = dinv
    for a in range(8):
        h = jnp.dot(x3_ref[:, a, :], w_ref[...],
                    preferred_element_type=jnp.float32)
        hs_ref[:, 16 * a:16 * (a + 1)] = h * dinv[:, 16 * a:16 * (a + 1)]


def _tc1(degp, x3, w):
    return pl.pallas_call(
        _tc1_body,
        out_shape=[
            jax.ShapeDtypeStruct((NROW, 128), jnp.float32),
            jax.ShapeDtypeStruct((NROW, 128), jnp.float32),
        ],
    )(degp, x3, w)


def _rep8(v):
    return jnp.concatenate([v] * 8, axis=0)


def _ln_relu(t, mavg_ref, g_ref, be_ref):
    hi = lax.Precision.HIGHEST
    mavg = mavg_ref[...]
    mu = jnp.dot(t, mavg, precision=hi, preferred_element_type=jnp.float32)
    d = t - mu
    var = jnp.dot(d * d, mavg, precision=hi,
                  preferred_element_type=jnp.float32)
    tn = d * lax.rsqrt(var + 1e-5) * _rep8(g_ref[...])[None, :] \
        + _rep8(be_ref[...])[None, :]
    return jnp.maximum(tn, 0.0)


def _tc_mid_body(aggp_ref, hs_ref, dinv_ref, b_ref, g_ref, be_ref,
                 wb_ref, mavg_ref, out_ref):
    dinv = dinv_ref[...]
    t = (aggp_ref[0:NROW, :] + aggp_ref[PPAD:PPAD + NROW, :] + hs_ref[...]) \
        * dinv + _rep8(b_ref[...])[None, :]
    r = _ln_relu(t, mavg_ref, g_ref, be_ref)
    out_ref[...] = jnp.dot(r, wb_ref[...],
                           preferred_element_type=jnp.float32) * dinv


def _tc_mid(aggp, hs, dinv, b, g, be, wb, mavg):
    return pl.pallas_call(
        _tc_mid_body,
        out_shape=jax.ShapeDtypeStruct((NROW, 128), jnp.float32),
    )(aggp, hs, dinv, b, g, be, wb, mavg)


def _tc_final_body(aggp_ref, hs_ref, dinv_ref, b_ref, g_ref, be_ref,
                   mavg_ref, f_ref, wl_ref, bl_ref, out_ref):
    hi = lax.Precision.HIGHEST
    t = (aggp_ref[0:NROW, :] + aggp_ref[PPAD:PPAD + NROW, :] + hs_ref[...]) \
        * dinv_ref[...] + _rep8(b_ref[...])[None, :]
    r = _ln_relu(t, mavg_ref, g_ref, be_ref)
    srow = jnp.sum(r, axis=0, keepdims=True)
    pooled = jnp.dot(srow, f_ref[...], precision=hi,
                     preferred_element_type=jnp.float32) * (1.0 / N)
    out_ref[...] = jnp.dot(pooled, wl_ref[...],
                           preferred_element_type=jnp.float32) \
        + bl_ref[...][None, :]


def _tc_final(aggp, hs, dinv, b, g, be, mavg, f, wl, bl):
    return pl.pallas_call(
        _tc_final_body,
        out_shape=jax.ShapeDtypeStruct((1, 25), jnp.float32),
    )(aggp, hs, dinv, b, g, be, mavg, f, wl, bl)


def kernel(x, edge_index, W1, b1, g1, be1, W2, b2, g2, be2, W3, b3, g3, be3,
           Wl, bl):
    e3 = edge_index.reshape(2, NW, G, C)
    x3 = x.reshape(NROW, 8, 128)
    eye8 = jnp.eye(8, dtype=jnp.float32)
    mavg = jnp.kron(eye8, jnp.full((16, 16), 1.0 / 16, jnp.float32))
    fmat = jnp.kron(jnp.ones((8, 1), jnp.float32),
                    jnp.eye(16, dtype=jnp.float32))

    degp = _deg_call(e3).reshape(NC * PPAD, 128)
    dinv, hs = _tc1(degp, x3, W1)

    aggp = _agg_call(hs.reshape(N, LANES), e3).reshape(NC * PPAD, 128)
    hs = _tc_mid(aggp, hs, dinv, b1, g1, be1, jnp.kron(eye8, W2), mavg)

    aggp = _agg_call(hs.reshape(N, LANES), e3).reshape(NC * PPAD, 128)
    hs = _tc_mid(aggp, hs, dinv, b2, g2, be2, jnp.kron(eye8, W3), mavg)

    aggp = _agg_call(hs.reshape(N, LANES), e3).reshape(NC * PPAD, 128)
    out = _tc_final(aggp, hs, dinv, b3, g3, be3, mavg, fmat, Wl, bl)
    return out.reshape(25)


# speedup vs baseline: 77.2338x; 1.2137x over previous
import functools

import jax
import jax.numpy as jnp
from jax import lax
from jax.experimental import pallas as pl
from jax.experimental.pallas import tpu as pltpu
from jax.experimental.pallas import tpu_sc as plsc

NC = 2
NS = 16
NW = NC * NS
LANES = 16

N = 10000
E = 320000
C = 80
EW = E // NW
G = EW // C
NPAD = 10240
RPS = NPAD // NS
NROW = N // 8
PPAD = NPAD // 8

NB = 10
PD = 5


def _sc_mesh():
    return plsc.VectorSubcoreMesh(
        core_axis_name="c", subcore_axis_name="s",
        num_cores=NC, num_subcores=NS)


def _deg_body(e_hbm, out_hbm, dst_slab, ones_v, stage, acc, sem):
    c = lax.axis_index("c")
    s = lax.axis_index("s")
    wid = c * NS + s
    pltpu.sync_copy(e_hbm.at[1].at[wid], dst_slab)

    def zrow(i, _):
        stage[i] = jnp.zeros((LANES,), jnp.float32)
        return 0
    lax.fori_loop(0, RPS, zrow, 0)

    def orow(i, _):
        ones_v[i] = jnp.ones((LANES,), jnp.float32)
        return 0
    lax.fori_loop(0, C, orow, 0)

    pltpu.sync_copy(stage, acc.at[pl.ds(s * RPS, RPS)])
    plsc.subcore_barrier()

    def drain():
        pltpu.make_async_copy(out_hbm.at[pl.ds(0, C)], ones_v, sem).wait()

    def chunk(g, _):
        pltpu.async_copy(ones_v, acc.at[dst_slab.at[g]], sem, add=True)

        @pl.when(g >= NB)
        def _():
            drain()
        return 0
    lax.fori_loop(0, G, chunk, 0)
    for _ in range(NB):
        drain()
    plsc.subcore_barrier()

    pltpu.sync_copy(acc.at[pl.ds(s * RPS, RPS)], stage)
    pltpu.sync_copy(stage, out_hbm.at[pl.ds(wid * RPS, RPS)])


@functools.lru_cache(maxsize=None)
def _deg_kernel():
    return pl.kernel(
        _deg_body,
        out_type=jax.ShapeDtypeStruct((NC * NPAD, LANES), jnp.float32),
        mesh=_sc_mesh(),
        scratch_types=[
            pltpu.VMEM((G, C), jnp.int32),
            pltpu.VMEM((C, LANES), jnp.float32),
            pltpu.VMEM((RPS, LANES), jnp.float32),
            pltpu.VMEM_SHARED((NPAD, LANES), jnp.float32),
            pltpu.SemaphoreType.DMA,
        ],
        compiler_params=pltpu.CompilerParams(use_tc_tiling_on_sc=False),
    )


def _deg_call(e3):
    return _deg_kernel()(e3)


def _agg_body(hs_hbm, e_hbm, out_hbm,
              src_slab, dst_slab, rows, stage, acc, gsem, ssem):
    c = lax.axis_index("c")
    s = lax.axis_index("s")
    wid = c * NS + s
    pltpu.sync_copy(e_hbm.at[0].at[wid], src_slab)
    pltpu.sync_copy(e_hbm.at[1].at[wid], dst_slab)

    def zrow(i, _):
        stage[i] = jnp.zeros((LANES,), jnp.float32)
        return 0
    lax.fori_loop(0, RPS, zrow, 0)
    pltpu.sync_copy(stage, acc.at[pl.ds(s * RPS, RPS)])
    plsc.subcore_barrier()

    def fire_gather(g):
        b = lax.rem(g, NB)
        pltpu.async_copy(hs_hbm.at[src_slab.at[g]], rows.at[b], gsem.at[b])

    def drain(sem_arr, b):
        pltpu.make_async_copy(hs_hbm.at[pl.ds(0, C)], rows.at[b],
                              sem_arr.at[b]).wait()

    for g in range(PD):
        fire_gather(jnp.int32(g))

    def chunk(g, _):
        b = lax.rem(g, NB)
        drain(gsem, b)
        pltpu.async_copy(rows.at[b], acc.at[dst_slab.at[g]], ssem.at[b],
                         add=True)
        gp = g + PD
        bp = lax.rem(gp, NB)

        @pl.when(gp < G)
        def _():
            @pl.when(gp >= NB)
            def _():
                drain(ssem, bp)
            fire_gather(gp)
        return 0
    lax.fori_loop(0, G, chunk, 0)
    for k in range(NB):
        drain(ssem, jnp.int32((G - NB + k) % NB))
    plsc.subcore_barrier()

    pltpu.sync_copy(acc.at[pl.ds(s * RPS, RPS)], stage)
    pltpu.sync_copy(stage, out_hbm.at[pl.ds(wid * RPS, RPS)])


@functools.lru_cache(maxsize=None)
def _agg_kernel():
    return pl.kernel(
        _agg_body,
        out_type=jax.ShapeDtypeStruct((NC * NPAD, LANES), jnp.float32),
        mesh=_sc_mesh(),
        scratch_types=[
            pltpu.VMEM((G, C), jnp.int32),
            pltpu.VMEM((G, C), jnp.int32),
            pltpu.VMEM((NB, C, LANES), jnp.float32),
            pltpu.VMEM((RPS, LANES), jnp.float32),
            pltpu.VMEM_SHARED((NPAD, LANES), jnp.float32),
            pltpu.SemaphoreType.DMA((NB,)),
            pltpu.SemaphoreType.DMA((NB,)),
        ],
        compiler_params=pltpu.CompilerParams(use_tc_tiling_on_sc=False),
    )


def _agg_call(hs, e3):
    return _agg_kernel()(hs, e3)


def _tc1_body(degp_ref, x3_ref, w_ref, dinv_ref, hs_ref):
    deg = degp_ref[0:NROW, :] + degp_ref[PPAD:PPAD + NROW, :] + 1.0
    dinv = lax.rsqrt(deg)
    dinv_ref[...] = dinv
    for a in range(8):
        h = jnp.dot(x3_ref[:, a, :], w_ref[...],
                    preferred_element_type=jnp.float32)
        hs_ref[:, 16 * a:16 * (a + 1)] = h * dinv[:, 16 * a:16 * (a + 1)]


def _tc1(degp, x3, w):
    return pl.pallas_call(
        _tc1_body,
        out_shape=[
            jax.ShapeDtypeStruct((NROW, 128), jnp.float32),
            jax.ShapeDtypeStruct((NROW, 128), jnp.float32),
        ],
    )(degp, x3, w)


def _rep8(v):
    return jnp.concatenate([v] * 8, axis=0)


def _ln_relu(t, mavg_ref, g_ref, be_ref):
    hi = lax.Precision.HIGHEST
    mavg = mavg_ref[...]
    mu = jnp.dot(t, mavg, precision=hi, preferred_element_type=jnp.float32)
    d = t - mu
    var = jnp.dot(d * d, mavg, precision=hi,
                  preferred_element_type=jnp.float32)
    tn = d * lax.rsqrt(var + 1e-5) * _rep8(g_ref[...])[None, :] \
        + _rep8(be_ref[...])[None, :]
    return jnp.maximum(tn, 0.0)


def _tc_mid_body(aggp_ref, hs_ref, dinv_ref, b_ref, g_ref, be_ref,
                 wb_ref, mavg_ref, out_ref):
    dinv = dinv_ref[...]
    t = (aggp_ref[0:NROW, :] + aggp_ref[PPAD:PPAD + NROW, :] + hs_ref[...]) \
        * dinv + _rep8(b_ref[...])[None, :]
    r = _ln_relu(t, mavg_ref, g_ref, be_ref)
    out_ref[...] = jnp.dot(r, wb_ref[...],
                           preferred_element_type=jnp.float32) * dinv


def _tc_mid(aggp, hs, dinv, b, g, be, wb, mavg):
    return pl.pallas_call(
        _tc_mid_body,
        out_shape=jax.ShapeDtypeStruct((NROW, 128), jnp.float32),
    )(aggp, hs, dinv, b, g, be, wb, mavg)


def _tc_final_body(aggp_ref, hs_ref, dinv_ref, b_ref, g_ref, be_ref,
                   mavg_ref, f_ref, wl_ref, bl_ref, out_ref):
    hi = lax.Precision.HIGHEST
    t = (aggp_ref[0:NROW, :] + aggp_ref[PPAD:PPAD + NROW, :] + hs_ref[...]) \
        * dinv_ref[...] + _rep8(b_ref[...])[None, :]
    r = _ln_relu(t, mavg_ref, g_ref, be_ref)
    srow = jnp.sum(r, axis=0, keepdims=True)
    pooled = jnp.dot(srow, f_ref[...], precision=hi,
                     preferred_element_type=jnp.float32) * (1.0 / N)
    out_ref[...] = jnp.dot(pooled, wl_ref[...],
                           preferred_element_type=jnp.float32) \
        + bl_ref[...][None, :]


def _tc_final(aggp, hs, dinv, b, g, be, mavg, f, wl, bl):
    return pl.pallas_call(
        _tc_final_body,
        out_shape=jax.ShapeDtypeStruct((1, 25), jnp.float32),
    )(aggp, hs, dinv, b, g, be, mavg, f, wl, bl)


def kernel(x, edge_index, W1, b1, g1, be1, W2, b2, g2, be2, W3, b3, g3, be3,
           Wl, bl):
    e3 = edge_index.reshape(2, NW, G, C)
    x3 = x.reshape(NROW, 8, 128)
    eye8 = jnp.eye(8, dtype=jnp.float32)
    mavg = jnp.kron(eye8, jnp.full((16, 16), 1.0 / 16, jnp.float32))
    fmat = jnp.kron(jnp.ones((8, 1), jnp.float32),
                    jnp.eye(16, dtype=jnp.float32))

    degp = _deg_call(e3).reshape(NC * PPAD, 128)
    dinv, hs = _tc1(degp, x3, W1)

    aggp = _agg_call(hs.reshape(N, LANES), e3).reshape(NC * PPAD, 128)
    hs = _tc_mid(aggp, hs, dinv, b1, g1, be1, jnp.kron(eye8, W2), mavg)

    aggp = _agg_call(hs.reshape(N, LANES), e3).reshape(NC * PPAD, 128)
    hs = _tc_mid(aggp, hs, dinv, b2, g2, be2, jnp.kron(eye8, W3), mavg)

    aggp = _agg_call(hs.reshape(N, LANES), e3).reshape(NC * PPAD, 128)
    out = _tc_final(aggp, hs, dinv, b3, g3, be3, mavg, fmat, Wl, bl)
    return out.reshape(25)


# speedup vs baseline: 86.3671x; 1.1183x over previous
import functools

import jax
import jax.numpy as jnp
from jax import lax
from jax.experimental import pallas as pl
from jax.experimental.pallas import tpu as pltpu
from jax.experimental.pallas import tpu_sc as plsc

NC = 2
NS = 16
NW = NC * NS
LANES = 16

N = 10000
E = 320000
C = 80
EW = E // NW
G = EW // C
NPAD = 10240
RPS = NPAD // NS
NROW = N // 8
PPAD = NPAD // 8

NB = 12
PD = 6


def _sc_mesh():
    return plsc.VectorSubcoreMesh(
        core_axis_name="c", subcore_axis_name="s",
        num_cores=NC, num_subcores=NS)


def _deg_body(e_hbm, out_hbm, dst_slab, ones_v, stage, acc, sem):
    c = lax.axis_index("c")
    s = lax.axis_index("s")
    wid = c * NS + s
    pltpu.sync_copy(e_hbm.at[1].at[wid], dst_slab)

    def zrow(i, _):
        stage[i] = jnp.zeros((LANES,), jnp.float32)
        return 0
    lax.fori_loop(0, RPS, zrow, 0)

    def orow(i, _):
        ones_v[i] = jnp.ones((LANES,), jnp.float32)
        return 0
    lax.fori_loop(0, C, orow, 0)

    pltpu.sync_copy(stage, acc.at[pl.ds(s * RPS, RPS)])
    plsc.subcore_barrier()

    def drain():
        pltpu.make_async_copy(out_hbm.at[pl.ds(0, C)], ones_v, sem).wait()

    def chunk(g, _):
        pltpu.async_copy(ones_v, acc.at[dst_slab.at[g]], sem, add=True)

        @pl.when(g >= NB)
        def _():
            drain()
        return 0
    lax.fori_loop(0, G, chunk, 0)
    for _ in range(NB):
        drain()
    plsc.subcore_barrier()

    pltpu.sync_copy(acc.at[pl.ds(s * RPS, RPS)], stage)
    pltpu.sync_copy(stage, out_hbm.at[pl.ds(wid * RPS, RPS)])


@functools.lru_cache(maxsize=None)
def _deg_kernel():
    return pl.kernel(
        _deg_body,
        out_type=jax.ShapeDtypeStruct((NC * NPAD, LANES), jnp.float32),
        mesh=_sc_mesh(),
        scratch_types=[
            pltpu.VMEM((G, C), jnp.int32),
            pltpu.VMEM((C, LANES), jnp.float32),
            pltpu.VMEM((RPS, LANES), jnp.float32),
            pltpu.VMEM_SHARED((NPAD, LANES), jnp.float32),
            pltpu.SemaphoreType.DMA,
        ],
        compiler_params=pltpu.CompilerParams(use_tc_tiling_on_sc=False),
    )


def _deg_call(e3):
    return _deg_kernel()(e3)


def _agg_body(hs_hbm, e_hbm, out_hbm,
              src_slab, dst_slab, rows, stage, acc, gsem, ssem, zsem):
    c = lax.axis_index("c")
    s = lax.axis_index("s")
    wid = c * NS + s
    src_load = pltpu.async_copy(e_hbm.at[0].at[wid], src_slab, zsem)
    dst_load = pltpu.async_copy(e_hbm.at[1].at[wid], dst_slab, zsem)

    def zrow(i, _):
        stage[i] = jnp.zeros((LANES,), jnp.float32)
        return 0
    lax.fori_loop(0, RPS, zrow, 0)
    src_load.wait()
    dst_load.wait()

    def fire_gather(g):
        b = lax.rem(g, NB)
        pltpu.async_copy(hs_hbm.at[src_slab.at[g]], rows.at[b], gsem.at[b])

    def drain(sem_arr, b):
        pltpu.make_async_copy(hs_hbm.at[pl.ds(0, C)], rows.at[b],
                              sem_arr.at[b]).wait()

    zero_copy = pltpu.async_copy(stage, acc.at[pl.ds(s * RPS, RPS)], zsem)
    for g in range(PD):
        fire_gather(jnp.int32(g))
    zero_copy.wait()
    plsc.subcore_barrier()

    def chunk(g, _):
        b = lax.rem(g, NB)
        drain(gsem, b)
        pltpu.async_copy(rows.at[b], acc.at[dst_slab.at[g]], ssem.at[b],
                         add=True)
        gp = g + PD
        bp = lax.rem(gp, NB)

        @pl.when(gp < G)
        def _():
            @pl.when(gp >= NB)
            def _():
                drain(ssem, bp)
            fire_gather(gp)
        return 0
    lax.fori_loop(0, G, chunk, 0)
    for k in range(NB):
        drain(ssem, jnp.int32((G - NB + k) % NB))
    plsc.subcore_barrier()

    pltpu.sync_copy(acc.at[pl.ds(s * RPS, RPS)], stage)
    pltpu.sync_copy(stage, out_hbm.at[pl.ds(wid * RPS, RPS)])


@functools.lru_cache(maxsize=None)
def _agg_kernel():
    return pl.kernel(
        _agg_body,
        out_type=jax.ShapeDtypeStruct((NC * NPAD, LANES), jnp.float32),
        mesh=_sc_mesh(),
        scratch_types=[
            pltpu.VMEM((G, C), jnp.int32),
            pltpu.VMEM((G, C), jnp.int32),
            pltpu.VMEM((NB, C, LANES), jnp.float32),
            pltpu.VMEM((RPS, LANES), jnp.float32),
            pltpu.VMEM_SHARED((NPAD, LANES), jnp.float32),
            pltpu.SemaphoreType.DMA((NB,)),
            pltpu.SemaphoreType.DMA((NB,)),
            pltpu.SemaphoreType.DMA,
        ],
        compiler_params=pltpu.CompilerParams(use_tc_tiling_on_sc=False),
    )


def _agg_call(hs, e3):
    return _agg_kernel()(hs, e3)


def _tc1_body(degp_ref, x3_ref, w_ref, dinv_ref, hs_ref):
    deg = degp_ref[0:NROW, :] + degp_ref[PPAD:PPAD + NROW, :] + 1.0
    dinv = lax.rsqrt(deg)
    dinv_ref[...] = dinv
    for a in range(8):
        h = jnp.dot(x3_ref[:, a, :], w_ref[...],
                    preferred_element_type=jnp.float32)
        hs_ref[:, 16 * a:16 * (a + 1)] = h * dinv[:, 16 * a:16 * (a + 1)]


def _tc1(degp, x3, w):
    return pl.pallas_call(
        _tc1_body,
        out_shape=[
            jax.ShapeDtypeStruct((NROW, 128), jnp.float32),
            jax.ShapeDtypeStruct((NROW, 128), jnp.float32),
        ],
    )(degp, x3, w)


def _rep8(v):
    return jnp.concatenate([v] * 8, axis=0)


def _ln_relu(t, mavg_ref, g_ref, be_ref):
    hi = lax.Precision.HIGHEST
    mavg = mavg_ref[...]
    mu = jnp.dot(t, mavg, precision=hi, preferred_element_type=jnp.float32)
    d = t - mu
    var = jnp.dot(d * d, mavg, precision=hi,
                  preferred_element_type=jnp.float32)
    tn = d * lax.rsqrt(var + 1e-5) * _rep8(g_ref[...])[None, :] \
        + _rep8(be_ref[...])[None, :]
    return jnp.maximum(tn, 0.0)


def _tc_mid_body(aggp_ref, hs_ref, dinv_ref, b_ref, g_ref, be_ref,
                 wb_ref, mavg_ref, out_ref):
    dinv = dinv_ref[...]
    t = (aggp_ref[0:NROW, :] + aggp_ref[PPAD:PPAD + NROW, :] + hs_ref[...]) \
        * dinv + _rep8(b_ref[...])[None, :]
    r = _ln_relu(t, mavg_ref, g_ref, be_ref)
    out_ref[...] = jnp.dot(r, wb_ref[...],
                           preferred_element_type=jnp.float32) * dinv


def _tc_mid(aggp, hs, dinv, b, g, be, wb, mavg):
    return pl.pallas_call(
        _tc_mid_body,
        out_shape=jax.ShapeDtypeStruct((NROW, 128), jnp.float32),
    )(aggp, hs, dinv, b, g, be, wb, mavg)


def _tc_final_body(aggp_ref, hs_ref, dinv_ref, b_ref, g_ref, be_ref,
                   mavg_ref, f_ref, wl_ref, bl_ref, out_ref):
    hi = lax.Precision.HIGHEST
    t = (aggp_ref[0:NROW, :] + aggp_ref[PPAD:PPAD + NROW, :] + hs_ref[...]) \
        * dinv_ref[...] + _rep8(b_ref[...])[None, :]
    r = _ln_relu(t, mavg_ref, g_ref, be_ref)
    srow = jnp.sum(r, axis=0, keepdims=True)
    pooled = jnp.dot(srow, f_ref[...], precision=hi,
                     preferred_element_type=jnp.float32) * (1.0 / N)
    out_ref[...] = jnp.dot(pooled, wl_ref[...],
                           preferred_element_type=jnp.float32) \
        + bl_ref[...][None, :]


def _tc_final(aggp, hs, dinv, b, g, be, mavg, f, wl, bl):
    return pl.pallas_call(
        _tc_final_body,
        out_shape=jax.ShapeDtypeStruct((1, 25), jnp.float32),
    )(aggp, hs, dinv, b, g, be, mavg, f, wl, bl)


def kernel(x, edge_index, W1, b1, g1, be1, W2, b2, g2, be2, W3, b3, g3, be3,
           Wl, bl):
    e3 = edge_index.reshape(2, NW, G, C)
    x3 = x.reshape(NROW, 8, 128)
    eye8 = jnp.eye(8, dtype=jnp.float32)
    mavg = jnp.kron(eye8, jnp.full((16, 16), 1.0 / 16, jnp.float32))
    fmat = jnp.kron(jnp.ones((8, 1), jnp.float32),
                    jnp.eye(16, dtype=jnp.float32))

    degp = _deg_call(e3).reshape(NC * PPAD, 128)
    dinv, hs = _tc1(degp, x3, W1)

    aggp = _agg_call(hs.reshape(N, LANES), e3).reshape(NC * PPAD, 128)
    hs = _tc_mid(aggp, hs, dinv, b1, g1, be1, jnp.kron(eye8, W2), mavg)

    aggp = _agg_call(hs.reshape(N, LANES), e3).reshape(NC * PPAD, 128)
    hs = _tc_mid(aggp, hs, dinv, b2, g2, be2, jnp.kron(eye8, W3), mavg)

    aggp = _agg_call(hs.reshape(N, LANES), e3).reshape(NC * PPAD, 128)
    out = _tc_final(aggp, hs, dinv, b3, g3, be3, mavg, fmat, Wl, bl)
    return out.reshape(25)
